# baseline (speedup 1.0000x reference)
import functools

import jax
import jax.numpy as jnp
import numpy as np
from jax.experimental import pallas as pl
from jax.experimental.pallas import tpu as pltpu

_L = 2048
_F = 1024
_E = 128
_TOPK = 22
_NBLK = 32


def _split_bf16(a):
    hi = a.astype(np.float32).astype(jnp.bfloat16)
    lo = (a.astype(np.float32) - np.asarray(hi, np.float32)).astype(jnp.bfloat16)
    return np.asarray(hi), np.asarray(lo)


def _consts():
    t = np.arange(_L)
    f = np.arange(_F)
    ang = 2.0 * np.pi * np.outer(t, f) / _L
    cs = np.concatenate(
        [np.cos(ang), np.sin(ang)], axis=1).astype(np.float32)
    cs_hi, cs_lo = _split_bf16(cs)
    wrow = np.full((1, _F), 2.0 / _L, np.float32)
    wrow[0, 0] = 1.0 / _L
    altcol = ((-1.0) ** t).astype(np.float32).reshape(_L, 1)
    pmat = (np.arange(_L)[None, :] % 8 == np.arange(8)[:, None]).astype(np.float32)
    return cs_hi, cs_lo, wrow, altcol, pmat


def _hi_lo(a):
    hi = a.astype(jnp.bfloat16)
    lo = (a - hi.astype(jnp.float32)).astype(jnp.bfloat16)
    return hi, lo


def _mm3(ahi, alo, bhi, blo, dims):
    dot = functools.partial(
        jax.lax.dot_general,
        dimension_numbers=dims,
        preferred_element_type=jnp.float32,
    )
    return dot(ahi, bhi) + dot(ahi, blo) + dot(alo, bhi)


def _body(q_ref, k_ref, v_ref, cs_hi_ref, cs_lo_ref, w_ref, alt_ref, p_ref,
          agg_ref, corr_ref, scr_ref):
    g = pl.program_id(0)

    @pl.when(g < _NBLK)
    def _corr_phase():
        q = q_ref[0]
        k = k_ref[0]
        cs_hi = cs_hi_ref[...]
        cs_lo = cs_lo_ref[...]
        wrow = w_ref[...]
        alt = alt_ref[...]

        cdims = (((0,), (0,)), ((), ()))
        q_hi, q_lo = _hi_lo(q)
        k_hi, k_lo = _hi_lo(k)
        qcs = _mm3(q_hi, q_lo, cs_hi, cs_lo, cdims)
        kcs = _mm3(k_hi, k_lo, cs_hi, cs_lo, cdims)
        qc, qs = qcs[:, :_F], qcs[:, _F:]
        kc, ks = kcs[:, :_F], kcs[:, _F:]

        re = (qc * kc + qs * ks) * wrow
        im = (qc * ks - qs * kc) * wrow
        rim = jnp.concatenate([re, -im], axis=1)
        rim_hi, rim_lo = _hi_lo(rim)
        fdims = (((1,), (1,)), ((), ()))
        corr = _mm3(cs_hi, cs_lo, rim_hi, rim_lo, fdims)

        qn = jnp.sum(q * alt, axis=0, keepdims=True)
        kn = jnp.sum(k * alt, axis=0, keepdims=True)
        corr = corr + alt * (qn * kn * (1.0 / _L))
        corr_ref[0] = corr
        scr_ref[pl.ds(g % 2, 1)] = corr[None]

    @pl.when(g > 0)
    def _select_phase():
        corr = scr_ref[pl.ds((g + 1) % 2, 1)][0]
        m_top = jnp.max(corr, axis=0, keepdims=True)
        arr16 = corr.astype(jnp.bfloat16)
        arr4 = arr16.reshape(_L // 16, 16, _E)
        neg = jnp.asarray(-jnp.inf, jnp.bfloat16)

        def _sel(_, thr):
            cand = jnp.where(arr4 < thr[None], arr4, neg)
            pmax = jnp.max(cand, axis=0)
            tmax = jnp.max(pmax, axis=0, keepdims=True)
            return jnp.broadcast_to(tmax, (16, _E))

        thr22 = jax.lax.fori_loop(
            0, _TOPK, _sel, jnp.full((16, _E), jnp.inf, jnp.bfloat16))

        sel = arr16 >= thr22[0:1]
        ex = jnp.exp(jnp.where(sel, corr - m_top, -1e4))
        s_grp = jnp.sum(ex.reshape(_L // 8, 8, _E), axis=0)
        z = jnp.sum(s_grp, axis=0, keepdims=True)
        s_grp = s_grp / z

        v8 = v_ref[0, 0:8, :]
        v16 = jnp.concatenate([v8, v8], axis=0)
        agg8 = jnp.zeros((8, _E), jnp.float32)
        for m in range(8):
            agg8 = agg8 + v16[m:m + 8, :] * s_grp[m:m + 1, :]
        agg_et = jax.lax.dot_general(
            agg8.T, p_ref[...],
            dimension_numbers=(((1,), (0,)), ((), ())),
            preferred_element_type=jnp.float32)
        agg_ref[0, 0] = agg_et


def kernel(queries, keys, values):
    b, l, h, e = queries.shape
    cs_hi, cs_lo, wrow, altcol, pmat = _consts()

    qf = queries.reshape(b, l, h * e)
    kf = keys.reshape(b, l, h * e)
    vf = values.reshape(b, l, h * e)

    def cur(g):
        gc = jnp.minimum(g, _NBLK - 1)
        return gc // h, gc % h

    def prev(g):
        gp = jnp.maximum(g - 1, 0)
        return gp // h, gp % h

    grid = (_NBLK + 1,)
    in_specs = [
        pl.BlockSpec((1, l, e), lambda g: (cur(g)[0], 0, cur(g)[1])),
        pl.BlockSpec((1, l, e), lambda g: (cur(g)[0], 0, cur(g)[1])),
        pl.BlockSpec((1, l, e), lambda g: (prev(g)[0], 0, prev(g)[1])),
        pl.BlockSpec((l, 2 * _F), lambda g: (0, 0)),
        pl.BlockSpec((l, 2 * _F), lambda g: (0, 0)),
        pl.BlockSpec((1, _F), lambda g: (0, 0)),
        pl.BlockSpec((l, 1), lambda g: (0, 0)),
        pl.BlockSpec((8, l), lambda g: (0, 0)),
    ]
    out_specs = [
        pl.BlockSpec((1, 1, e, l),
                     lambda g: (prev(g)[0], prev(g)[1], 0, 0)),
        pl.BlockSpec((1, l, e), lambda g: (cur(g)[0], 0, cur(g)[1])),
    ]
    out_shapes = [
        jax.ShapeDtypeStruct((b, h, e, l), jnp.float32),
        jax.ShapeDtypeStruct((b, l, h * e), jnp.float32),
    ]
    agg, corr = pl.pallas_call(
        _body,
        grid=grid,
        in_specs=in_specs,
        out_specs=out_specs,
        out_shape=out_shapes,
        scratch_shapes=[pltpu.VMEM((2, l, e), jnp.float32)],
    )(qf, kf, vf, cs_hi, cs_lo, wrow, altcol, pmat)
    return agg, corr.reshape(b, l, h, e)


# speedup vs baseline: 1.2908x; 1.2908x over previous
import functools

import jax
import jax.numpy as jnp
import numpy as np
from jax.experimental import pallas as pl
from jax.experimental.pallas import tpu as pltpu

_L = 2048
_F = 1024
_EB = 256
_TOPK = 22


def _split_bf16(a):
    hi = a.astype(np.float32).astype(jnp.bfloat16)
    lo = (a.astype(np.float32) - np.asarray(hi, np.float32)).astype(jnp.bfloat16)
    return np.asarray(hi), np.asarray(lo)


def _consts():
    t = np.arange(_L)
    f = np.arange(_F)
    ang = 2.0 * np.pi * np.outer(t, f) / _L
    cs = np.concatenate(
        [np.cos(ang), np.sin(ang)], axis=1).astype(np.float32)
    cs_hi, cs_lo = _split_bf16(cs)
    wrow = np.full((1, _F), 2.0 / _L, np.float32)
    wrow[0, 0] = 1.0 / _L
    pmat = (np.arange(_L)[None, :] % 8 == np.arange(8)[:, None]).astype(np.float32)
    return cs_hi, cs_lo, wrow, pmat


def _hi_lo(a):
    hi = a.astype(jnp.bfloat16)
    lo = (a - hi.astype(jnp.float32)).astype(jnp.bfloat16)
    return hi, lo


def _mm3(ahi, alo, bhi, blo, dims):
    dot = functools.partial(
        jax.lax.dot_general,
        dimension_numbers=dims,
        preferred_element_type=jnp.float32,
    )
    return dot(ahi, bhi) + dot(ahi, blo) + dot(alo, bhi)


def _body(q_ref, k_ref, v_ref, cs_hi_ref, cs_lo_ref, w_ref, p_ref,
          agg_ref, corr_ref):
    q = q_ref[0]
    k = k_ref[0]
    cs_hi = cs_hi_ref[...]
    cs_lo = cs_lo_ref[...]
    wrow = w_ref[...]
    sgn = 1.0 - 2.0 * (jax.lax.broadcasted_iota(
        jnp.int32, (1, 8, 1), 1) % 2).astype(jnp.float32)

    cdims = (((0,), (0,)), ((), ()))
    q_hi, q_lo = _hi_lo(q)
    k_hi, k_lo = _hi_lo(k)
    qcs = _mm3(q_hi, q_lo, cs_hi, cs_lo, cdims)
    kcs = _mm3(k_hi, k_lo, cs_hi, cs_lo, cdims)
    qc, qs = qcs[:, :_F], qcs[:, _F:]
    kc, ks = kcs[:, :_F], kcs[:, _F:]

    re = (qc * kc + qs * ks) * wrow
    im = (qc * ks - qs * kc) * wrow
    rim = jnp.concatenate([re, -im], axis=1)
    rim_hi, rim_lo = _hi_lo(rim)
    fdims = (((1,), (1,)), ((), ()))
    corr = _mm3(cs_hi, cs_lo, rim_hi, rim_lo, fdims)

    q8 = jnp.sum(q.reshape(_L // 8, 8, _EB), axis=0)
    k8 = jnp.sum(k.reshape(_L // 8, 8, _EB), axis=0)
    qn = jnp.sum(q8 * sgn[0], axis=0, keepdims=True)
    kn = jnp.sum(k8 * sgn[0], axis=0, keepdims=True)
    nyq = qn * kn * (1.0 / _L)
    corr = (corr.reshape(_L // 8, 8, _EB) + sgn * nyq[None]).reshape(_L, _EB)
    corr_ref[0] = corr

    m_top = jnp.max(corr, axis=0, keepdims=True)
    arr16 = corr.astype(jnp.bfloat16)
    arr4 = arr16.reshape(_L // 16, 16, _EB)
    neg = jnp.asarray(-jnp.inf, jnp.bfloat16)

    def _sel(_, thr):
        cand = jnp.where(arr4 < thr[None], arr4, neg)
        pmax = jnp.max(cand, axis=0)
        tmax = jnp.max(pmax, axis=0, keepdims=True)
        return jnp.broadcast_to(tmax, (16, _EB))

    thr22 = jax.lax.fori_loop(
        0, _TOPK, _sel, jnp.full((16, _EB), jnp.inf, jnp.bfloat16))

    sel = arr16 >= thr22[0:1]
    ex = jnp.exp(jnp.where(sel, corr - m_top, -1e4))
    s_grp = jnp.sum(ex.reshape(_L // 8, 8, _EB), axis=0)
    z = jnp.sum(s_grp, axis=0, keepdims=True)
    s_grp = s_grp / z

    v8 = v_ref[0, 0:8, :]
    v16 = jnp.concatenate([v8, v8], axis=0)
    agg8 = jnp.zeros((8, _EB), jnp.float32)
    for m in range(8):
        agg8 = agg8 + v16[m:m + 8, :] * s_grp[m:m + 1, :]
    agg_et = jax.lax.dot_general(
        agg8.T, p_ref[...],
        dimension_numbers=(((1,), (0,)), ((), ())),
        preferred_element_type=jnp.float32)
    agg_ref[0] = agg_et.reshape(2, _EB // 2, _L)


def kernel(queries, keys, values):
    b, l, h, e = queries.shape
    cs_hi, cs_lo, wrow, pmat = _consts()

    qf = queries.reshape(b, l, h * e)
    kf = keys.reshape(b, l, h * e)
    vf = values.reshape(b, l, h * e)

    grid = (b, h // 2)
    eb = 2 * e
    in_specs = [
        pl.BlockSpec((1, l, eb), lambda i, j: (i, 0, j)),
        pl.BlockSpec((1, l, eb), lambda i, j: (i, 0, j)),
        pl.BlockSpec((1, l, eb), lambda i, j: (i, 0, j)),
        pl.BlockSpec((l, 2 * _F), lambda i, j: (0, 0)),
        pl.BlockSpec((l, 2 * _F), lambda i, j: (0, 0)),
        pl.BlockSpec((1, _F), lambda i, j: (0, 0)),
        pl.BlockSpec((8, l), lambda i, j: (0, 0)),
    ]
    out_specs = [
        pl.BlockSpec((1, 2, e, l), lambda i, j: (i, j, 0, 0)),
        pl.BlockSpec((1, l, eb), lambda i, j: (i, 0, j)),
    ]
    out_shapes = [
        jax.ShapeDtypeStruct((b, h, e, l), jnp.float32),
        jax.ShapeDtypeStruct((b, l, h * e), jnp.float32),
    ]
    agg, corr = pl.pallas_call(
        _body,
        grid=grid,
        in_specs=in_specs,
        out_specs=out_specs,
        out_shape=out_shapes,
        compiler_params=pltpu.CompilerParams(
            dimension_semantics=("parallel", "parallel"),
            vmem_limit_bytes=64 * 1024 * 1024),
    )(qf, kf, vf, cs_hi, cs_lo, wrow, pmat)
    return agg, corr.reshape(b, l, h, e)
